# Initial kernel scaffold; baseline (speedup 1.0000x reference)
import jax
import jax.numpy as jnp
from jax.experimental import pallas as pl


def kernel(signal, f0_upsampled, phase_cycles):
    raise NotImplementedError("write your pallas kernel here")


# speedup vs baseline: 32.5523x; 32.5523x over previous
import jax
import jax.numpy as jnp
import math
from jax.experimental import pallas as pl

_SAMPLE_RATE = 24000
_NYQUIST = _SAMPLE_RATE / 2.0
_R = 0.92
_R2 = _R * _R
_POWER_FACTOR = 0.1
_EPSILON = 1e-6
_TWO_LOG2_R = 2.0 * math.log2(_R)
_AMP_NUM = 2.0 * (1.0 - _R2) / _R2

_NEG_INF = float("-inf")


def _shift_fwd(x, d, sentinel):
    pad = jnp.full(x.shape[:-1] + (d,), sentinel, dtype=x.dtype)
    return jnp.concatenate([pad, x[..., :-d]], axis=-1)


def _shift_bwd(x, d, sentinel):
    pad = jnp.full(x.shape[:-1] + (d,), sentinel, dtype=x.dtype)
    return jnp.concatenate([x[..., d:], pad], axis=-1)


def _tc_body(sig_ref, f0_ref, ph_ref, out_ref, pulse_ref):
    sig = sig_ref[...]
    f0 = f0_ref[...]
    ph = ph_ref[...]

    voiced = f0 > 1.0
    safe_f0 = jnp.maximum(f0, 1e-5)
    n_harm = jnp.floor(_NYQUIST / safe_f0)
    r2n = jnp.exp2(n_harm * _TWO_LOG2_R)
    amp = _POWER_FACTOR * jnp.sqrt(_AMP_NUM / jnp.maximum(1.0 - r2n, _EPSILON))
    out = sig * amp * voiced.astype(jnp.float32)
    out_ref[...] = out

    v = jnp.where(voiced, out, _NEG_INF)
    k = ph.astype(jnp.int32)

    T = v.shape[-1]
    n_steps = max(1, (T - 1).bit_length())

    m = v
    for t in range(n_steps):
        d = 1 << t
        ks = _shift_fwd(k, d, -1)
        ms = _shift_fwd(m, d, _NEG_INF)
        m = jnp.where(k == ks, jnp.maximum(m, ms), m)
    fwd = m

    k1 = _shift_fwd(k, 1, -1)
    f1 = _shift_fwd(fwd, 1, _NEG_INF)
    fexcl = jnp.where(k == k1, f1, _NEG_INF)

    m = v
    for t in range(n_steps):
        d = 1 << t
        ks = _shift_bwd(k, d, -1)
        ms = _shift_bwd(m, d, _NEG_INF)
        m = jnp.where(k == ks, jnp.maximum(m, ms), m)
    bwd = m

    pulse = (v > fexcl) & (bwd <= v)
    pulse_ref[...] = pulse.astype(jnp.int32)


def kernel(signal, f0_upsampled, phase_cycles):
    B = signal.shape[0]
    T = signal.shape[-1]
    sig2 = signal.reshape(B, T)
    f02 = f0_upsampled.reshape(B, T)
    ph2 = phase_cycles.reshape(B, T)

    out2, pulse2 = pl.pallas_call(
        _tc_body,
        out_shape=(
            jax.ShapeDtypeStruct((B, T), jnp.float32),
            jax.ShapeDtypeStruct((B, T), jnp.int32),
        ),
    )(sig2, f02, ph2)

    out = out2.reshape(signal.shape)
    pulse_locs = pulse2.astype(jnp.bool_).reshape(signal.shape)
    return out, pulse_locs
